# baseline (speedup 1.0000x reference)
import functools
import math

import jax
import jax.numpy as jnp
from jax import lax
from jax.experimental import pallas as pl
from jax.experimental.pallas import tpu as pltpu
from jax.experimental.pallas import tpu_sc as plsc

EMB = 128
LANES = 16
CHUNK = 128
NBUF = 2
NC, NS = 2, 16
NW = NC * NS

_SCALE = math.sqrt(EMB)


def _make_lookup(total_rows: int):
  assert total_rows % (NW * CHUNK) == 0
  chunks_per_w = total_rows // (NW * CHUNK)
  assert chunks_per_w % NBUF == 0
  n_steps = chunks_per_w // NBUF

  mesh = plsc.VectorSubcoreMesh(core_axis_name="c", subcore_axis_name="s")

  @functools.partial(
      pl.kernel,
      out_type=jax.ShapeDtypeStruct((total_rows, EMB), jnp.float32),
      mesh=mesh,
      scratch_types=(
          [pltpu.VMEM((chunks_per_w, CHUNK), jnp.int32)]
          + [pltpu.VMEM((CHUNK, EMB), jnp.float32)] * (2 * NBUF)
          + [pltpu.SemaphoreType.DMA] * (2 * NBUF)
      ),
  )
  def lookup(tok_hbm, table_hbm, out_hbm, idx_all,
             in0, in1, ob0, ob1, gs0, gs1, os0, os1):
    in_bufs = [in0, in1]
    out_bufs = [ob0, ob1]
    gsems = [gs0, gs1]
    osems = [os0, os1]

    wid = lax.axis_index("s") * NC + lax.axis_index("c")
    base_chunk = wid * chunks_per_w

    pltpu.sync_copy(tok_hbm.at[pl.ds(base_chunk, chunks_per_w)], idx_all)

    for b in range(NBUF):
      pltpu.async_copy(table_hbm.at[idx_all.at[b]], in_bufs[b], gsems[b])

    def step(i, carry):
      for b in range(NBUF):
        c = i * NBUF + b

        @pl.when(i > 0)
        def _wait_out():
          pltpu.make_async_copy(
              out_bufs[b], out_hbm.at[pl.ds(0, CHUNK)], osems[b]).wait()

        pltpu.make_async_copy(
            table_hbm.at[pl.ds(0, CHUNK)], in_bufs[b], gsems[b]).wait()

        @plsc.parallel_loop(0, CHUNK, step=1, unroll=4)
        def _scale_row(r):
          for k in range(EMB // LANES):
            sl = pl.ds(k * LANES, LANES)
            out_bufs[b][r, sl] = in_bufs[b][r, sl] * _SCALE

        @pl.when(c + NBUF < chunks_per_w)
        def _prefetch():
          pltpu.async_copy(
              table_hbm.at[idx_all.at[c + NBUF]], in_bufs[b], gsems[b])

        pltpu.async_copy(
            out_bufs[b],
            out_hbm.at[pl.ds((base_chunk + c) * CHUNK, CHUNK)],
            osems[b])
      return carry

    lax.fori_loop(0, n_steps, step, 0)

    for b in range(NBUF):
      pltpu.make_async_copy(
          out_bufs[b], out_hbm.at[pl.ds(0, CHUNK)], osems[b]).wait()

  return lookup


def kernel(tokens, table):
  n_tok = tokens.size
  tok2d = tokens.reshape(-1).astype(jnp.int32).reshape(n_tok // CHUNK, CHUNK)
  out = _make_lookup(n_tok)(tok2d, table)
  return out.reshape(*tokens.shape, EMB)


# speedup vs baseline: 1.0073x; 1.0073x over previous
import functools
import math

import jax
import jax.numpy as jnp
from jax import lax
from jax.experimental import pallas as pl
from jax.experimental.pallas import tpu as pltpu
from jax.experimental.pallas import tpu_sc as plsc

EMB = 128
LANES = 16
CHUNK = 128
NBUF = 3
NC, NS = 2, 16
NW = NC * NS

_SCALE = math.sqrt(EMB)


def _make_lookup(total_rows: int):
  assert total_rows % (NW * CHUNK) == 0
  chunks_per_w = total_rows // (NW * CHUNK)
  n_steps = chunks_per_w // NBUF
  n_tail = chunks_per_w - n_steps * NBUF

  mesh = plsc.VectorSubcoreMesh(core_axis_name="c", subcore_axis_name="s")

  @functools.partial(
      pl.kernel,
      out_type=jax.ShapeDtypeStruct((total_rows, EMB), jnp.float32),
      mesh=mesh,
      scratch_types=(
          [pltpu.VMEM((chunks_per_w, CHUNK), jnp.int32)]
          + [pltpu.VMEM((CHUNK, EMB), jnp.float32)] * (2 * NBUF)
          + [pltpu.SemaphoreType.DMA] * (2 * NBUF)
      ),
  )
  def lookup(tok_hbm, table_hbm, out_hbm, idx_all, *bufs_and_sems):
    in_bufs = bufs_and_sems[:NBUF]
    out_bufs = bufs_and_sems[NBUF:2 * NBUF]
    gsems = bufs_and_sems[2 * NBUF:3 * NBUF]
    osems = bufs_and_sems[3 * NBUF:]

    wid = lax.axis_index("s") * NC + lax.axis_index("c")
    base_chunk = wid * chunks_per_w

    pltpu.sync_copy(tok_hbm.at[pl.ds(base_chunk, chunks_per_w)], idx_all)

    def wait_gather(b):
      pltpu.make_async_copy(
          table_hbm.at[pl.ds(0, CHUNK)], in_bufs[b], gsems[b]).wait()

    def wait_out(b):
      pltpu.make_async_copy(
          out_bufs[b], out_hbm.at[pl.ds(0, CHUNK)], osems[b]).wait()

    def start_gather(c, b):
      pltpu.async_copy(table_hbm.at[idx_all.at[c]], in_bufs[b], gsems[b])

    def start_out(c, b):
      pltpu.async_copy(
          out_bufs[b], out_hbm.at[pl.ds((base_chunk + c) * CHUNK, CHUNK)],
          osems[b])

    def scale(b):
      @plsc.parallel_loop(0, CHUNK, step=1, unroll=4)
      def _scale_row(r):
        for k in range(EMB // LANES):
          sl = pl.ds(k * LANES, LANES)
          out_bufs[b][r, sl] = in_bufs[b][r, sl] * _SCALE

    for b in range(NBUF):
      start_gather(b, b)

    def step(i, carry):
      for b in range(NBUF):
        c = i * NBUF + b

        @pl.when(i > 0)
        def _wait_out():
          wait_out(b)

        wait_gather(b)
        scale(b)

        @pl.when(c + NBUF < chunks_per_w)
        def _prefetch():
          start_gather(c + NBUF, b)

        start_out(c, b)
      return carry

    lax.fori_loop(0, n_steps, step, 0)

    for t in range(n_tail):
      cc = n_steps * NBUF + t
      b = cc % NBUF
      wait_out(b)
      wait_gather(b)
      scale(b)
      start_out(cc, b)

    for b in range(NBUF):
      wait_out(b)

  return lookup


def kernel(tokens, table):
  n_tok = tokens.size
  tok2d = tokens.reshape(-1).astype(jnp.int32).reshape(n_tok // CHUNK, CHUNK)
  out = _make_lookup(n_tok)(tok2d, table)
  return out.reshape(*tokens.shape, EMB)
